# Initial kernel scaffold; baseline (speedup 1.0000x reference)
import jax
import jax.numpy as jnp
from jax.experimental import pallas as pl


def kernel(pred, label):
    raise NotImplementedError("write your pallas kernel here")


# speedup vs baseline: 2.2993x; 2.2993x over previous
import functools

import jax
import jax.numpy as jnp
from jax import lax
from jax.experimental import pallas as pl
from jax.experimental.pallas import tpu as pltpu
from jax.experimental.pallas import tpu_sc as plsc

GH = 52
GW = 52
G = GH * GW
NA = 5
GA = G * NA
NC = 5
ROW = GA * NC
ANCW = (0.05, 0.11, 0.2, 0.35, 0.7)
ANCH = (0.07, 0.15, 0.3, 0.5, 0.8)

_SC_WORKERS = 16


def _build_consts(bn):
    j = jnp.arange(ROW, dtype=jnp.int32)
    c = j % NC
    ga = j // NC
    g = ga // NA
    a = ga % NA
    gx = ((g // GH).astype(jnp.float32) + 0.5) / GW
    gy = ((g % GH).astype(jnp.float32) + 0.5) / GH
    aw = jnp.array(ANCW, jnp.float32)[a]
    ah = jnp.array(ANCH, jnp.float32)[a]
    is_exp = (c == 2) | (c == 3)
    s = jnp.where(c == 2, aw, ah)
    t = jnp.where(c == 0, gx,
        jnp.where(c == 1, gy,
        jnp.where(c == 2, aw,
        jnp.where(c == 3, ah, 0.0)))).astype(jnp.float32)
    wcoor = 1.0 / (bn * (GA - 1) * 4)
    wconf = 1.0 / (bn * (GA - 1))
    w = jnp.where(c == 4, wconf, wcoor).astype(jnp.float32)
    ms = jnp.where(is_exp, s, 0.0).astype(jnp.float32)
    q = jnp.where(is_exp, 0.0, 1.0).astype(jnp.float32)
    rs = lambda x: x.reshape(1, ROW)
    return rs(ms), rs(q), rs(t), rs(w)


def _dense_body(p_ref, ms_ref, q_ref, t_ref, w_ref, out_ref):
    i = pl.program_id(0)

    @pl.when(i == 0)
    def _():
        out_ref[...] = jnp.zeros_like(out_ref)

    p = p_ref[...]
    ms = ms_ref[...]
    q = q_ref[...]
    sgn = 1.0 - 2.0 * q
    e = jnp.exp(p * sgn)
    r = 1.0 / (1.0 + e)
    v = ms * e + q * r
    d = v - t_ref[...]
    part = jnp.sum(w_ref[...] * d * d, axis=1, keepdims=True)
    out_ref[...] += jnp.sum(part, axis=0, keepdims=True)


def _dense_sum(pred2d, bb):
    bn = pred2d.shape[0]
    ms, q, t, w = _build_consts(bn)
    cspec = pl.BlockSpec((1, ROW), lambda i: (0, 0))
    return pl.pallas_call(
        _dense_body,
        grid=(bn // bb,),
        in_specs=[pl.BlockSpec((bb, ROW), lambda i: (i, 0)),
                  cspec, cspec, cspec, cspec],
        out_specs=pl.BlockSpec((1, 1), lambda i: (0, 0)),
        out_shape=jax.ShapeDtypeStruct((1, 1), jnp.float32),
    )(pred2d, ms, q, t, w)


def _sc_match_gather(label1d, pred1d, bn):
    spw = bn // _SC_WORKERS
    info = plsc.get_sparse_core_info()
    ncores = info.num_cores
    mesh = plsc.VectorSubcoreMesh(core_axis_name="c", subcore_axis_name="s")

    @functools.partial(
        pl.kernel, mesh=mesh,
        out_type=[jax.ShapeDtypeStruct((bn,), jnp.int32),
                  jax.ShapeDtypeStruct((NC * bn,), jnp.float32)],
        scratch_types=[pltpu.VMEM((4 * spw,), jnp.float32),
                       pltpu.VMEM((spw,), jnp.int32),
                       pltpu.VMEM((NC * spw,), jnp.int32),
                       pltpu.VMEM((NC * spw,), jnp.float32),
                       pltpu.SemaphoreType.DMA],
    )
    def sc_kernel(lab_hbm, pred_hbm, fi_hbm, obj_hbm,
                  lab_v, fi_v, idx_v, objs_v, sem):
        wid = lax.axis_index("s") * ncores + lax.axis_index("c")

        @pl.when(wid < _SC_WORKERS)
        def _():
            base = wid * spw
            for c in range(4):
                pltpu.sync_copy(lab_hbm.at[pl.ds(c * bn + base, spw)],
                                lab_v.at[pl.ds(c * spw, spw)])
            lx = lab_v[pl.ds(0 * spw, spw)]
            ly = lab_v[pl.ds(1 * spw, spw)]
            lw = lab_v[pl.ds(2 * spw, spw)]
            lh = lab_v[pl.ds(3 * spw, spw)]
            ix = (lx * GW).astype(jnp.int32)
            iy = (ly * GH).astype(jnp.int32)
            dw0 = lw - ANCW[0]
            dh0 = lh - ANCH[0]
            bd = dw0 * dw0 + dh0 * dh0
            ba = jnp.zeros((spw,), jnp.int32)
            for k in range(1, NA):
                dwk = lw - ANCW[k]
                dhk = lh - ANCH[k]
                dk = dwk * dwk + dhk * dhk
                upd = dk > bd
                ba = jnp.where(upd, k, ba)
                bd = jnp.where(upd, dk, bd)
            fi = (ix * GW + iy) * NA + ba
            elem0 = ((base + lax.iota(jnp.int32, spw)) * GA + fi) * NC
            for c in range(NC):
                idx_v[pl.ds(c * spw, spw)] = elem0 + c
            pltpu.async_copy(pred_hbm.at[idx_v], objs_v, sem).wait()
            for c in range(NC):
                pltpu.sync_copy(objs_v.at[pl.ds(c * spw, spw)],
                                obj_hbm.at[pl.ds(c * bn + base, spw)])
            fi_v[...] = fi
            pltpu.sync_copy(fi_v, fi_hbm.at[pl.ds(base, spw)])

    return sc_kernel(label1d, pred1d)


def _epi_body(sw_ref, obj_ref, fi_ref, lab_ref, out_ref):
    bn = fi_ref.shape[1]
    fi = fi_ref[...]
    idx = fi // NA
    a = fi - idx * NA
    ixg = idx // GH
    iyg = idx - ixg * GH
    gxt = (ixg.astype(jnp.float32) + 0.5) / GW
    gyt = (iyg.astype(jnp.float32) + 0.5) / GH
    aw = jnp.full(a.shape, ANCW[0], jnp.float32)
    ah = jnp.full(a.shape, ANCH[0], jnp.float32)
    for k in range(1, NA):
        aw = jnp.where(a == k, ANCW[k], aw)
        ah = jnp.where(a == k, ANCH[k], ah)
    px = jax.nn.sigmoid(obj_ref[0:1, :])
    py = jax.nn.sigmoid(obj_ref[1:2, :])
    pw = jnp.exp(obj_ref[2:3, :]) * aw
    ph = jnp.exp(obj_ref[3:4, :]) * ah
    pc = jax.nn.sigmoid(obj_ref[4:5, :])
    o_coor = (px - gxt) ** 2 + (py - gyt) ** 2 + (pw - aw) ** 2 + (ph - ah) ** 2
    o_conf = pc * pc
    wcoor = 1.0 / (bn * (GA - 1) * 4)
    wconf = 1.0 / (bn * (GA - 1))
    ssum = lambda x: jnp.sum(x, axis=1, keepdims=True)
    sub = ssum(o_coor) * wcoor + ssum(o_conf) * wconf
    lx = lab_ref[0:1, :]
    ly = lab_ref[1:2, :]
    lw = lab_ref[2:3, :]
    lh = lab_ref[3:4, :]
    coor_obj = ssum((px - lx) ** 2 + (py - ly) ** 2
                    + (pw - lw) ** 2 + (ph - lh) ** 2) / (bn * 4)
    lx0 = jnp.maximum(lx - lw * 0.5, 0.0)
    ly0 = jnp.maximum(ly - lh * 0.5, 0.0)
    lx1 = jnp.minimum(lx + lw * 0.5, 1.0)
    ly1 = jnp.minimum(ly + lh * 0.5, 1.0)
    px0 = jnp.maximum(px - pw * 0.5, 0.0)
    py0 = jnp.maximum(py - ph * 0.5, 0.0)
    px1 = jnp.minimum(px + pw * 0.5, 1.0)
    py1 = jnp.minimum(py + ph * 0.5, 1.0)
    ix0 = jnp.maximum(lx0, px0)
    iy0 = jnp.maximum(ly0, py0)
    ix1 = jnp.minimum(lx1, px1)
    iy1 = jnp.minimum(ly1, py1)
    la = lx1 * ly1
    pa = px1 * py1
    ia = jnp.maximum(ix1 - ix0, 0.0) * jnp.maximum(iy1 - iy0, 0.0)
    iou = ia / (la + pa - ia)
    conf_obj = ssum((pc - iou) ** 2) / bn
    out_ref[...] = sw_ref[...] - sub + coor_obj + conf_obj


def _epilogue(s_w, obj_t, fi2, lab_t):
    bn = fi2.shape[1]
    return pl.pallas_call(
        _epi_body,
        in_specs=[pl.BlockSpec((1, 1), lambda: (0, 0)),
                  pl.BlockSpec((NC, bn), lambda: (0, 0)),
                  pl.BlockSpec((1, bn), lambda: (0, 0)),
                  pl.BlockSpec((4, bn), lambda: (0, 0))],
        out_specs=pl.BlockSpec((1, 1), lambda: (0, 0)),
        out_shape=jax.ShapeDtypeStruct((1, 1), jnp.float32),
    )(s_w, obj_t, fi2, lab_t)


def kernel(pred, label):
    bn = pred.shape[0]
    pred2d = pred.reshape(bn, ROW)
    pred1d = pred.reshape(bn * ROW)
    label_t = label.T
    fi, obj = _sc_match_gather(label_t.reshape(4 * bn), pred1d, bn)
    s_w = _dense_sum(pred2d, 8)
    out = _epilogue(s_w, obj.reshape(NC, bn), fi.reshape(1, bn), label_t)
    return out[0, 0]
